# Initial kernel scaffold; baseline (speedup 1.0000x reference)
import jax
import jax.numpy as jnp
from jax.experimental import pallas as pl


def kernel(x, w_qkv, b_qkv, w_o, b_o, w1, b1, w2, b2, gamma1, beta1, gamma2, beta2):
    raise NotImplementedError("write your pallas kernel here")


# speedup vs baseline: 2.1759x; 2.1759x over previous
import functools
import math

import jax
import jax.numpy as jnp
from jax.experimental import pallas as pl
from jax.experimental.pallas import tpu as pltpu

_NUM_HEADS = 12
_EPS = 1e-5


def _layernorm_f32(x, g, b, inv_d):
    s1 = jnp.sum(x, axis=-1, keepdims=True)
    s2 = jnp.sum(x * x, axis=-1, keepdims=True)
    mean = s1 * inv_d
    var = s2 * inv_d - mean * mean
    inv_std = jax.lax.rsqrt(var + _EPS)
    scale = g * inv_std
    shift = b - mean * scale
    return x * scale + shift


def _attn_kernel(x_ref, wqkv_ref, bqkv_ref, o_ref, *, seq, d_model):
    hd = d_model // _NUM_HEADS
    sm_scale = 1.0 / math.sqrt(hd)

    xb = x_ref[...].astype(jnp.bfloat16)
    qkv = jnp.dot(xb, wqkv_ref[...], preferred_element_type=jnp.float32)
    qkv = qkv + bqkv_ref[...]

    for h in range(_NUM_HEADS):
        base = h * 3 * hd
        qh = qkv[:, base:base + hd].astype(jnp.bfloat16)
        kh = qkv[:, base + hd:base + 2 * hd].astype(jnp.bfloat16)
        vh = qkv[:, base + 2 * hd:base + 3 * hd].astype(jnp.bfloat16)
        s = jax.lax.dot_general(
            qh, kh, (((1,), (1,)), ((), ())),
            preferred_element_type=jnp.float32) * sm_scale
        s = s - jnp.max(s, axis=-1, keepdims=True)
        p = jnp.exp(s)
        p = p / jnp.sum(p, axis=-1, keepdims=True)
        oh = jnp.dot(p.astype(jnp.bfloat16), vh,
                     preferred_element_type=jnp.float32)
        o_ref[0, h * seq:(h + 1) * seq, :] = oh.astype(jnp.bfloat16)


def _ffn_kernel(v_ref, x_ref, wo_ref, bo_ref, w1_ref, b1_ref,
                w2_ref, b2_ref, g1_ref, bt1_ref, g2_ref, bt2_ref, o_ref,
                *, d_model):
    inv_d = 1.0 / d_model
    attn = jnp.dot(v_ref[...], wo_ref[...],
                   preferred_element_type=jnp.float32) + bo_ref[...]
    h1 = _layernorm_f32(attn + x_ref[...], g1_ref[...], bt1_ref[...], inv_d)

    ff = jnp.dot(h1.astype(jnp.bfloat16), w1_ref[...],
                 preferred_element_type=jnp.float32) + b1_ref[...]
    ff = jnp.maximum(ff, 0.0)
    ff2 = jnp.dot(ff.astype(jnp.bfloat16), w2_ref[...],
                  preferred_element_type=jnp.float32) + b2_ref[...]
    o_ref[...] = _layernorm_f32(ff2 + h1, g2_ref[...], bt2_ref[...], inv_d)


def kernel(x, w_qkv, b_qkv, w_o, b_o, w1, b1, w2, b2,
           gamma1, beta1, gamma2, beta2):
    b, s, d = x.shape
    dff = w1.shape[1]
    hd = d // _NUM_HEADS
    rows = b * s
    x2 = x.reshape(rows, d)

    wqkv_b = w_qkv.astype(jnp.bfloat16)
    wo_b = w_o.astype(jnp.bfloat16)
    w1_b = w1.astype(jnp.bfloat16)
    w2_b = w2.astype(jnp.bfloat16)

    cparams = pltpu.CompilerParams(
        dimension_semantics=("parallel",),
        vmem_limit_bytes=100 * 1024 * 1024,
    )

    def const(shape):
        return pl.BlockSpec(shape, lambda i: (0,) * len(shape))

    vals = pl.pallas_call(
        functools.partial(_attn_kernel, seq=s, d_model=d),
        out_shape=jax.ShapeDtypeStruct((b, _NUM_HEADS * s, hd), jnp.bfloat16),
        grid=(b,),
        in_specs=[
            pl.BlockSpec((s, d), lambda i: (i, 0)),
            const((d, 3 * d)),
            const((1, 3 * d)),
        ],
        out_specs=pl.BlockSpec((1, _NUM_HEADS * s, hd), lambda i: (i, 0, 0)),
        compiler_params=cparams,
    )(x2, wqkv_b, b_qkv.reshape(1, 3 * d))

    vals2 = vals.reshape(rows, d)

    row_tile = 256 if rows % 256 == 0 else rows
    out = pl.pallas_call(
        functools.partial(_ffn_kernel, d_model=d),
        out_shape=jax.ShapeDtypeStruct((rows, d), jnp.float32),
        grid=(rows // row_tile,),
        in_specs=[
            pl.BlockSpec((row_tile, d), lambda i: (i, 0)),
            pl.BlockSpec((row_tile, d), lambda i: (i, 0)),
            const((d, d)),
            const((1, d)),
            const((d, dff)),
            const((1, dff)),
            const((dff, d)),
            const((1, d)),
            const((1, d)),
            const((1, d)),
            const((1, d)),
            const((1, d)),
        ],
        out_specs=pl.BlockSpec((row_tile, d), lambda i: (i, 0)),
        compiler_params=cparams,
    )(vals2, x2, wo_b, b_o.reshape(1, d),
      w1_b, b1.reshape(1, dff), w2_b, b2.reshape(1, d),
      gamma1.reshape(1, d), beta1.reshape(1, d),
      gamma2.reshape(1, d), beta2.reshape(1, d))
    return out.reshape(b, s, d)


# speedup vs baseline: 3.1007x; 1.4250x over previous
import functools
import math

import jax
import jax.numpy as jnp
from jax.experimental import pallas as pl
from jax.experimental.pallas import tpu as pltpu

_NUM_HEADS = 12
_EPS = 1e-5


def _layernorm_f32(x, g, b, inv_d):
    s1 = jnp.sum(x, axis=-1, keepdims=True)
    s2 = jnp.sum(x * x, axis=-1, keepdims=True)
    mean = s1 * inv_d
    var = s2 * inv_d - mean * mean
    inv_std = jax.lax.rsqrt(var + _EPS)
    scale = g * inv_std
    shift = b - mean * scale
    return x * scale + shift


def _attn_kernel(x_ref, wqkv_ref, bqkv_ref, o_ref, *, seq, d_model):
    hd = d_model // _NUM_HEADS
    sm_scale = 1.0 / math.sqrt(hd)

    xb = x_ref[...].astype(jnp.bfloat16)
    qkv = jnp.dot(xb, wqkv_ref[...], preferred_element_type=jnp.float32)
    qkv = qkv + bqkv_ref[...]

    scores = []
    vs = []
    for h in range(_NUM_HEADS):
        base = h * 3 * hd
        qh = qkv[:, base:base + hd].astype(jnp.bfloat16)
        kh = qkv[:, base + hd:base + 2 * hd].astype(jnp.bfloat16)
        vs.append(qkv[:, base + 2 * hd:base + 3 * hd].astype(jnp.bfloat16))
        scores.append(jax.lax.dot_general(
            qh, kh, (((1,), (1,)), ((), ())),
            preferred_element_type=jnp.float32) * sm_scale)
    probs = []
    for h in range(_NUM_HEADS):
        s = scores[h]
        s = s - jnp.max(s, axis=-1, keepdims=True)
        p = jnp.exp(s)
        p = p / jnp.sum(p, axis=-1, keepdims=True)
        probs.append(p.astype(jnp.bfloat16))
    for h in range(_NUM_HEADS):
        oh = jnp.dot(probs[h], vs[h],
                     preferred_element_type=jnp.float32)
        o_ref[0, h * seq:(h + 1) * seq, :] = oh.astype(jnp.bfloat16)


def _ffn_kernel(v_ref, x_ref, wo_ref, bo_ref, w1_ref, b1_ref,
                w2_ref, b2_ref, g1_ref, bt1_ref, g2_ref, bt2_ref, o_ref,
                *, d_model):
    inv_d = 1.0 / d_model
    attn = jnp.dot(v_ref[...], wo_ref[...],
                   preferred_element_type=jnp.float32) + bo_ref[...]
    h1 = _layernorm_f32(attn + x_ref[...], g1_ref[...], bt1_ref[...], inv_d)

    ff = jnp.dot(h1.astype(jnp.bfloat16), w1_ref[...],
                 preferred_element_type=jnp.float32) + b1_ref[...]
    ff = jnp.maximum(ff, 0.0)
    ff2 = jnp.dot(ff.astype(jnp.bfloat16), w2_ref[...],
                  preferred_element_type=jnp.float32) + b2_ref[...]
    o_ref[...] = _layernorm_f32(ff2 + h1, g2_ref[...], bt2_ref[...], inv_d)


def kernel(x, w_qkv, b_qkv, w_o, b_o, w1, b1, w2, b2,
           gamma1, beta1, gamma2, beta2):
    b, s, d = x.shape
    dff = w1.shape[1]
    hd = d // _NUM_HEADS
    rows = b * s
    x2 = x.reshape(rows, d)

    wqkv_b = w_qkv.astype(jnp.bfloat16)
    wo_b = w_o.astype(jnp.bfloat16)
    w1_b = w1.astype(jnp.bfloat16)
    w2_b = w2.astype(jnp.bfloat16)

    cparams = pltpu.CompilerParams(
        dimension_semantics=("parallel",),
        vmem_limit_bytes=100 * 1024 * 1024,
    )

    def const(shape):
        return pl.BlockSpec(shape, lambda i: (0,) * len(shape))

    vals = pl.pallas_call(
        functools.partial(_attn_kernel, seq=s, d_model=d),
        out_shape=jax.ShapeDtypeStruct((b, _NUM_HEADS * s, hd), jnp.bfloat16),
        grid=(b,),
        in_specs=[
            pl.BlockSpec((s, d), lambda i: (i, 0)),
            const((d, 3 * d)),
            const((1, 3 * d)),
        ],
        out_specs=pl.BlockSpec((1, _NUM_HEADS * s, hd), lambda i: (i, 0, 0)),
        compiler_params=cparams,
    )(x2, wqkv_b, b_qkv.reshape(1, 3 * d))

    vals2 = vals.reshape(rows, d)

    row_tile = 512 if rows % 512 == 0 else rows
    out = pl.pallas_call(
        functools.partial(_ffn_kernel, d_model=d),
        out_shape=jax.ShapeDtypeStruct((rows, d), jnp.float32),
        grid=(rows // row_tile,),
        in_specs=[
            pl.BlockSpec((row_tile, d), lambda i: (i, 0)),
            pl.BlockSpec((row_tile, d), lambda i: (i, 0)),
            const((d, d)),
            const((1, d)),
            const((d, dff)),
            const((1, dff)),
            const((dff, d)),
            const((1, d)),
            const((1, d)),
            const((1, d)),
            const((1, d)),
            const((1, d)),
        ],
        out_specs=pl.BlockSpec((row_tile, d), lambda i: (i, 0)),
        compiler_params=cparams,
    )(vals2, x2, wo_b, b_o.reshape(1, d),
      w1_b, b1.reshape(1, dff), w2_b, b2.reshape(1, d),
      gamma1.reshape(1, d), beta1.reshape(1, d),
      gamma2.reshape(1, d), beta2.reshape(1, d))
    return out.reshape(b, s, d)
